# baseline (speedup 1.0000x reference)
import jax
import jax.numpy as jnp
from jax.experimental import pallas as pl
from jax.experimental.pallas import tpu as pltpu

_BB = 16


def _body(x_ref, c_ref, o_ref):
    d = o_ref.shape[1]
    for bb in range(x_ref.shape[0]):
        x = x_ref[bb]
        e = jnp.exp(x - jnp.max(x, axis=0, keepdims=True))
        proj = jax.lax.dot_general(
            c_ref[...], e, (((0,), (0,)), ((), ())),
            preferred_element_type=jnp.float32,
        )
        o_ref[bb] = proj[:d] * (1.0 / proj[d:d + 1])


def kernel(base_logits, compatibility_matrix):
    b, seq, vocab = base_logits.shape
    diac = compatibility_matrix.shape[1]

    xt = jnp.transpose(base_logits, (0, 2, 1))
    caug = jnp.concatenate(
        [compatibility_matrix, jnp.ones((vocab, 1), jnp.float32)], axis=1
    )
    out_t = pl.pallas_call(
        _body,
        grid=(b // _BB,),
        in_specs=[
            pl.BlockSpec((_BB, vocab, seq), lambda i: (i, 0, 0)),
            pl.BlockSpec((vocab, diac + 1), lambda i: (0, 0)),
        ],
        out_specs=pl.BlockSpec((_BB, diac, seq), lambda i: (i, 0, 0)),
        out_shape=jax.ShapeDtypeStruct((b, diac, seq), jnp.float32),
        compiler_params=pltpu.CompilerParams(
            dimension_semantics=("parallel",),
        ),
    )(xt, caug)
    return jnp.transpose(out_t, (0, 2, 1))
